# baseline (speedup 1.0000x reference)
import jax
import jax.numpy as jnp
from jax.experimental import pallas as pl
from jax.experimental.pallas import tpu as pltpu

_SIZE = 1000
_NCLS = 10
_NPTS = 8
_BR = 200
_NB = _SIZE // _BR


def _viz_kernel(z_ref, lab_ref, out_ref):
    blk = pl.program_id(0)
    row0 = blk * _BR
    out_ref[...] = jnp.zeros_like(out_ref)
    col_ids = jax.lax.broadcasted_iota(jnp.int32, (1, _SIZE * _NCLS), 1)
    for n in range(_NPTS):
        a = z_ref[n, 0] * _SIZE
        b = z_ref[n, 1] * _SIZE
        j = a.astype(jnp.int32)
        i = b.astype(jnp.int32)
        valid = (a > j.astype(jnp.float32)) & (b > i.astype(jnp.float32))
        k = i * _NCLS + lab_ref[n]
        in_block = valid & (j >= row0) & (j < row0 + _BR)

        @pl.when(in_block)
        def _():
            jl = j - row0
            row = (col_ids == k).astype(jnp.float32)
            cur = out_ref[pl.ds(jl, 1), :]
            out_ref[pl.ds(jl, 1), :] = jnp.maximum(cur, row)


def kernel(z, labels):
    labels = labels.astype(jnp.int32)
    out = pl.pallas_call(
        _viz_kernel,
        out_shape=jax.ShapeDtypeStruct((_SIZE, _SIZE * _NCLS), jnp.float32),
        grid=(_NB,),
        in_specs=[
            pl.BlockSpec(memory_space=pltpu.SMEM),
            pl.BlockSpec(memory_space=pltpu.SMEM),
        ],
        out_specs=pl.BlockSpec((_BR, _SIZE * _NCLS), lambda m: (m, 0)),
        compiler_params=pltpu.CompilerParams(
            dimension_semantics=("parallel",),
        ),
        name="visual_imitation",
    )(z, labels)
    return out.reshape(_SIZE, _SIZE, _NCLS)
